# baseline (speedup 1.0000x reference)
import functools

import jax
import jax.numpy as jnp
from jax import lax
from jax.experimental import pallas as pl
from jax.experimental.pallas import tpu as pltpu
from jax.experimental.pallas import tpu_sc as plsc

NUM_CORES = 2
NUM_SUBCORES = 16
NW = NUM_CORES * NUM_SUBCORES

NBUF = 4
SPLIT = 96


def _gather_fn(B, H, D, V, DP):
    BW = B // NW
    assert B % NW == 0 and BW % NBUF == 0 and SPLIT % 8 == 0
    n_steps = BW // NBUF

    mesh = plsc.VectorSubcoreMesh(
        core_axis_name="c", subcore_axis_name="s",
        num_cores=NUM_CORES, num_subcores=NUM_SUBCORES)

    @functools.partial(
        pl.kernel,
        out_type=jax.ShapeDtypeStruct((B, H, DP), jnp.float32),
        mesh=mesh,
        scratch_types=[
            pltpu.VMEM((BW, H), jnp.int32),
            pltpu.VMEM((NBUF, H, DP), jnp.float32),
            [pltpu.SemaphoreType.DMA] * NBUF,
            [pltpu.SemaphoreType.DMA] * NBUF,
        ],
        compiler_params=pltpu.CompilerParams(
            use_tc_tiling_on_sc=False, needs_layout_passes=False),
    )
    def gather_kernel(idx_hbm, table_hbm, out_hbm, idx_v, rows_v,
                      sem_g, sem_w):
        wid = lax.axis_index("s") * NUM_CORES + lax.axis_index("c")
        b0 = wid * BW
        pltpu.sync_copy(idx_hbm.at[pl.ds(b0, BW)], idx_v)

        def start_gather(b, nb):
            pltpu.async_copy(
                table_hbm.at[idx_v.at[b, pl.ds(0, SPLIT)]],
                rows_v.at[nb, pl.ds(0, SPLIT)], sem_g[nb])
            pltpu.async_copy(
                table_hbm.at[idx_v.at[b, pl.ds(SPLIT, H - SPLIT)]],
                rows_v.at[nb, pl.ds(SPLIT, H - SPLIT)], sem_g[nb])

        def wait_gather(nb):
            pltpu.make_async_copy(
                table_hbm.at[idx_v.at[0, pl.ds(0, SPLIT)]],
                rows_v.at[nb, pl.ds(0, SPLIT)], sem_g[nb]).wait()
            pltpu.make_async_copy(
                table_hbm.at[idx_v.at[0, pl.ds(SPLIT, H - SPLIT)]],
                rows_v.at[nb, pl.ds(SPLIT, H - SPLIT)], sem_g[nb]).wait()

        def start_write(b, nb):
            pltpu.async_copy(rows_v.at[nb], out_hbm.at[b0 + b], sem_w[nb])

        def wait_write(nb):
            pltpu.make_async_copy(
                rows_v.at[nb], out_hbm.at[b0], sem_w[nb]).wait()

        for nb in range(NBUF):
            start_gather(nb, nb)

        def body(g, carry):
            for nb in range(NBUF):
                b = g * NBUF + nb
                wait_gather(nb)
                start_write(b, nb)

                @pl.when(g < n_steps - 1)
                def _():
                    wait_write(nb)
                    start_gather(b + NBUF, nb)

            return carry

        lax.fori_loop(0, n_steps, body, 0, unroll=False)

        for nb in range(NBUF):
            wait_write(nb)

    return gather_kernel


def _pad_transpose_fn(V, D, DP, CB=512):
    grid = (V + CB - 1) // CB

    def body(in_ref, out_ref):
        out_ref[:, 0:D] = in_ref[...].T

    return pl.pallas_call(
        body,
        grid=(grid,),
        in_specs=[pl.BlockSpec((D, CB), lambda j: (0, j))],
        out_specs=pl.BlockSpec((CB, DP), lambda j: (j, 0)),
        out_shape=jax.ShapeDtypeStruct((V, DP), jnp.float32),
    )


def kernel(input_, weight):
    B, H = input_.shape
    V, D = weight.shape
    DP = 128
    idx = input_.astype(jnp.int32)
    wt_pad = _pad_transpose_fn(V, D, DP)(weight.T)
    out = _gather_fn(B, H, D, V, DP)(idx, wt_pad)
    return out[:, :, :D]


# speedup vs baseline: 1.6667x; 1.6667x over previous
import functools

import jax
import jax.numpy as jnp
from jax import lax
from jax.experimental import pallas as pl
from jax.experimental.pallas import tpu as pltpu
from jax.experimental.pallas import tpu_sc as plsc

NUM_CORES = 2
NUM_SUBCORES = 16
NW = NUM_CORES * NUM_SUBCORES

NBUF = 4
SPLIT = 96


def _gather_fn(B, H, D, V, DP):
    BW = B // NW
    assert B % NW == 0 and BW % NBUF == 0 and SPLIT % 8 == 0
    n_steps = BW // NBUF

    mesh = plsc.VectorSubcoreMesh(
        core_axis_name="c", subcore_axis_name="s",
        num_cores=NUM_CORES, num_subcores=NUM_SUBCORES)

    @functools.partial(
        pl.kernel,
        out_type=jax.ShapeDtypeStruct((B, H, DP), jnp.float32),
        mesh=mesh,
        scratch_types=[
            pltpu.VMEM((BW * H,), jnp.int32),
            pltpu.VMEM((NBUF, H, DP), jnp.float32),
            [pltpu.SemaphoreType.DMA] * NBUF,
            [pltpu.SemaphoreType.DMA] * NBUF,
        ],
        compiler_params=pltpu.CompilerParams(
            use_tc_tiling_on_sc=False, needs_layout_passes=False),
    )
    def gather_kernel(idx_hbm, table_hbm, out_hbm, idx_v, rows_v,
                      sem_g, sem_w):
        wid = lax.axis_index("s") * NUM_CORES + lax.axis_index("c")
        b0 = wid * BW
        pltpu.sync_copy(idx_hbm.at[pl.ds(b0 * H, BW * H)], idx_v)

        def start_gather(b, nb):
            pltpu.async_copy(
                table_hbm.at[idx_v.at[pl.ds(b * H, SPLIT)]],
                rows_v.at[nb, pl.ds(0, SPLIT)], sem_g[nb])
            pltpu.async_copy(
                table_hbm.at[idx_v.at[pl.ds(b * H + SPLIT, H - SPLIT)]],
                rows_v.at[nb, pl.ds(SPLIT, H - SPLIT)], sem_g[nb])

        def wait_gather(nb):
            pltpu.make_async_copy(
                table_hbm.at[idx_v.at[pl.ds(0, SPLIT)]],
                rows_v.at[nb, pl.ds(0, SPLIT)], sem_g[nb]).wait()
            pltpu.make_async_copy(
                table_hbm.at[idx_v.at[pl.ds(SPLIT, H - SPLIT)]],
                rows_v.at[nb, pl.ds(SPLIT, H - SPLIT)], sem_g[nb]).wait()

        def start_write(b, nb):
            pltpu.async_copy(rows_v.at[nb], out_hbm.at[b0 + b], sem_w[nb])

        def wait_write(nb):
            pltpu.make_async_copy(
                rows_v.at[nb], out_hbm.at[b0], sem_w[nb]).wait()

        for nb in range(NBUF):
            start_gather(nb, nb)

        def body(g, carry):
            for nb in range(NBUF):
                b = g * NBUF + nb
                wait_gather(nb)
                start_write(b, nb)

                @pl.when(g < n_steps - 1)
                def _():
                    wait_write(nb)
                    start_gather(b + NBUF, nb)

            return carry

        lax.fori_loop(0, n_steps, body, 0, unroll=False)

        for nb in range(NBUF):
            wait_write(nb)

    return gather_kernel


def _pad_transpose_fn(V, D, DP, CB=512):
    grid = (V + CB - 1) // CB

    def body(in_ref, out_ref):
        out_ref[:, 0:D] = in_ref[...].T

    return pl.pallas_call(
        body,
        grid=(grid,),
        in_specs=[pl.BlockSpec((D, CB), lambda j: (0, j))],
        out_specs=pl.BlockSpec((CB, DP), lambda j: (j, 0)),
        out_shape=jax.ShapeDtypeStruct((V, DP), jnp.float32),
    )


def kernel(input_, weight):
    B, H = input_.shape
    V, D = weight.shape
    DP = 128
    idx = input_.reshape(B * H).astype(jnp.int32)
    wt_pad = jnp.pad(weight, ((0, 0), (0, DP - D)))
    out = _gather_fn(B, H, D, V, DP)(idx, wt_pad)
    return out[:, :, :D]


# speedup vs baseline: 1.7988x; 1.0793x over previous
import functools

import jax
import jax.numpy as jnp
from jax import lax
from jax.experimental import pallas as pl
from jax.experimental.pallas import tpu as pltpu
from jax.experimental.pallas import tpu_sc as plsc

NUM_CORES = 2
NUM_SUBCORES = 16
NW = NUM_CORES * NUM_SUBCORES

NBUF = 4
SPLIT = 96


def _gather_fn(B, H, D, V, DP):
    BW = B // NW
    assert B % NW == 0 and BW % NBUF == 0 and SPLIT % 8 == 0
    n_steps = BW // NBUF

    mesh = plsc.VectorSubcoreMesh(
        core_axis_name="c", subcore_axis_name="s",
        num_cores=NUM_CORES, num_subcores=NUM_SUBCORES)

    @functools.partial(
        pl.kernel,
        out_type=jax.ShapeDtypeStruct((B, H, DP), jnp.float32),
        mesh=mesh,
        scratch_types=[
            pltpu.VMEM((BW * H,), jnp.int32),
            pltpu.VMEM((NBUF, H, DP), jnp.float32),
            [pltpu.SemaphoreType.DMA] * NBUF,
            [pltpu.SemaphoreType.DMA] * NBUF,
        ],
        compiler_params=pltpu.CompilerParams(
            use_tc_tiling_on_sc=False, needs_layout_passes=False),
    )
    def gather_kernel(idx_hbm, table_hbm, out_hbm, idx_v, rows_v,
                      sem_g, sem_w):
        wid = lax.axis_index("s") * NUM_CORES + lax.axis_index("c")
        b0 = wid * BW
        pltpu.sync_copy(idx_hbm.at[pl.ds(b0 * H, BW * H)], idx_v)

        def start_gather(b, nb):
            pltpu.async_copy(
                table_hbm.at[idx_v.at[pl.ds(b * H, SPLIT)]],
                rows_v.at[nb, pl.ds(0, SPLIT)], sem_g[nb])
            pltpu.async_copy(
                table_hbm.at[idx_v.at[pl.ds(b * H + SPLIT, H - SPLIT)]],
                rows_v.at[nb, pl.ds(SPLIT, H - SPLIT)], sem_g[nb])

        def wait_gather(nb):
            pltpu.make_async_copy(
                table_hbm.at[idx_v.at[pl.ds(0, SPLIT)]],
                rows_v.at[nb, pl.ds(0, SPLIT)], sem_g[nb]).wait()
            pltpu.make_async_copy(
                table_hbm.at[idx_v.at[pl.ds(SPLIT, H - SPLIT)]],
                rows_v.at[nb, pl.ds(SPLIT, H - SPLIT)], sem_g[nb]).wait()

        def start_write(b, nb):
            pltpu.async_copy(
                rows_v.at[nb, :, pl.ds(0, 64)],
                out_hbm.at[b0 + b, :, pl.ds(0, 64)], sem_w[nb])

        def wait_write(nb):
            pltpu.make_async_copy(
                rows_v.at[nb, :, pl.ds(0, 64)],
                out_hbm.at[b0, :, pl.ds(0, 64)], sem_w[nb]).wait()

        for nb in range(NBUF):
            start_gather(nb, nb)

        def body(g, carry):
            for nb in range(NBUF):
                b = g * NBUF + nb
                wait_gather(nb)
                start_write(b, nb)

                @pl.when(g < n_steps - 1)
                def _():
                    wait_write(nb)
                    start_gather(b + NBUF, nb)

            return carry

        lax.fori_loop(0, n_steps, body, 0, unroll=False)

        for nb in range(NBUF):
            wait_write(nb)

    return gather_kernel


def _pad_transpose_fn(V, D, DP, CB=512):
    grid = (V + CB - 1) // CB

    def body(in_ref, out_ref):
        out_ref[:, 0:D] = in_ref[...].T

    return pl.pallas_call(
        body,
        grid=(grid,),
        in_specs=[pl.BlockSpec((D, CB), lambda j: (0, j))],
        out_specs=pl.BlockSpec((CB, DP), lambda j: (j, 0)),
        out_shape=jax.ShapeDtypeStruct((V, DP), jnp.float32),
    )


def kernel(input_, weight):
    B, H = input_.shape
    V, D = weight.shape
    DP = 128
    idx = input_.reshape(B * H).astype(jnp.int32)
    wt_pad = jnp.pad(weight, ((0, 0), (0, DP - D)))
    out = _gather_fn(B, H, D, V, DP)(idx, wt_pad)
    return out[:, :, :D]


# speedup vs baseline: 1.8014x; 1.0014x over previous
import functools

import jax
import jax.numpy as jnp
from jax import lax
from jax.experimental import pallas as pl
from jax.experimental.pallas import tpu as pltpu
from jax.experimental.pallas import tpu_sc as plsc

NUM_CORES = 2
NUM_SUBCORES = 16
NW = NUM_CORES * NUM_SUBCORES

NBUF = 4
SPLIT = 96


def _gather_fn(B, H, D, V, DP):
    BW = B // NW
    assert B % NW == 0 and BW % NBUF == 0 and SPLIT % 8 == 0
    n_steps = BW // NBUF

    mesh = plsc.VectorSubcoreMesh(
        core_axis_name="c", subcore_axis_name="s",
        num_cores=NUM_CORES, num_subcores=NUM_SUBCORES)

    @functools.partial(
        pl.kernel,
        out_type=jax.ShapeDtypeStruct((B, H, DP), jnp.float32),
        mesh=mesh,
        scratch_types=[
            pltpu.VMEM((BW * H,), jnp.int32),
            pltpu.VMEM((NBUF, H, DP), jnp.float32),
            [pltpu.SemaphoreType.DMA] * NBUF,
            [pltpu.SemaphoreType.DMA] * NBUF,
        ],
        compiler_params=pltpu.CompilerParams(
            use_tc_tiling_on_sc=False, needs_layout_passes=False),
    )
    def gather_kernel(idx_hbm, table_hbm, out_hbm, idx_v, rows_v,
                      sem_g, sem_w):
        wid = lax.axis_index("s") * NUM_CORES + lax.axis_index("c")
        b0 = wid * BW
        pltpu.sync_copy(idx_hbm.at[pl.ds(b0 * H, BW * H)], idx_v)

        def start_gather(b, nb):
            pltpu.async_copy(
                table_hbm.at[idx_v.at[pl.ds(b * H, SPLIT)]],
                rows_v.at[nb, pl.ds(0, SPLIT)], sem_g[nb])
            pltpu.async_copy(
                table_hbm.at[idx_v.at[pl.ds(b * H + SPLIT, H - SPLIT)]],
                rows_v.at[nb, pl.ds(SPLIT, H - SPLIT)], sem_g[nb])

        def wait_gather(nb):
            pltpu.make_async_copy(
                table_hbm.at[idx_v.at[pl.ds(0, SPLIT)]],
                rows_v.at[nb, pl.ds(0, SPLIT)], sem_g[nb]).wait()
            pltpu.make_async_copy(
                table_hbm.at[idx_v.at[pl.ds(SPLIT, H - SPLIT)]],
                rows_v.at[nb, pl.ds(SPLIT, H - SPLIT)], sem_g[nb]).wait()

        def start_write(b, nb):
            pltpu.async_copy(
                rows_v.at[nb, :, pl.ds(0, 64)],
                out_hbm.at[b0 + b, :, pl.ds(0, 64)], sem_w[nb])

        def wait_write(nb):
            pltpu.make_async_copy(
                rows_v.at[nb, :, pl.ds(0, 64)],
                out_hbm.at[b0, :, pl.ds(0, 64)], sem_w[nb]).wait()

        for nb in range(NBUF):
            start_gather(nb, nb)

        def body(g, carry):
            for nb in range(NBUF):
                b = g * NBUF + nb
                wait_gather(nb)
                start_write(b, nb)

                @pl.when(g < n_steps - 1)
                def _():
                    wait_write(nb)
                    start_gather(b + NBUF, nb)

            return carry

        lax.fori_loop(0, n_steps, body, 0, unroll=False)

        for nb in range(NBUF):
            wait_write(nb)

    return gather_kernel


def kernel(input_, weight):
    B, H = input_.shape
    V, D = weight.shape
    DP = 128
    idx = input_.reshape(B * H).astype(jnp.int32)
    wt_pad = jnp.pad(weight, ((0, 0), (0, DP - D)))
    out = _gather_fn(B, H, D, V, DP)(idx, wt_pad)
    return out[:, :, :D]
